# baseline (speedup 1.0000x reference)
import functools

import jax
import jax.numpy as jnp
from jax import lax
from jax.experimental import pallas as pl
from jax.experimental.pallas import tpu as pltpu
from jax.experimental.pallas import tpu_sc as plsc

NUM_HEADS = 4
EMBED_DIM = 256
NUM_EMB = 8192
HEAD_DIM = EMBED_DIM // NUM_HEADS
BATCH = 16384

BB = 128
NQ = 4
QB = BB // NQ
ROW_W = NUM_HEADS * NUM_EMB


def _normalize_cb_body(cb_ref, cbn_ref):
    cb = cb_ref[...]
    cb_sq = jnp.sum(cb * cb, axis=-1, keepdims=True)
    cbn_ref[...] = cb / jnp.maximum(jnp.sqrt(cb_sq), 1e-12)


def _normalize_cb(codebooks):
    return pl.pallas_call(
        _normalize_cb_body,
        out_shape=jax.ShapeDtypeStruct(
            (NUM_HEADS, NUM_EMB, HEAD_DIM), jnp.float32),
    )(codebooks)


def _dist_argmax_body(z_ref, cbn_ref, d0_ref, d1_ref, d2_ref, d3_ref,
                      idx_ref, fidx_ref):
    d_refs = (d0_ref, d1_ref, d2_ref, d3_ref)
    zb = z_ref[...]
    for h in range(NUM_HEADS):
        zh = zb[:, h * HEAD_DIM:(h + 1) * HEAD_DIM]
        zn_sq = jnp.sum(zh * zh, axis=-1, keepdims=True)
        zn = zh / jnp.maximum(jnp.sqrt(zn_sq), 1e-12)
        sims = lax.dot_general(
            zn, cbn_ref[h], (((1,), (1,)), ((), ())),
            preferred_element_type=jnp.float32)
        dist = 1.0 - sims
        for q in range(NQ):
            d_refs[q][:, h * NUM_EMB:(h + 1) * NUM_EMB] = (
                dist[q * QB:(q + 1) * QB, :])
        idx = jnp.argmax(sims, axis=-1).astype(jnp.int32)
        idx_ref[h, 0, :] = idx
        fidx_ref[h, 0, :] = idx + h * NUM_EMB


def _dist_argmax(z_perm, cbn):
    grid = (BATCH // BB,)
    quarter = jax.ShapeDtypeStruct((BATCH // NQ, ROW_W), jnp.float32)
    return pl.pallas_call(
        _dist_argmax_body,
        grid=grid,
        in_specs=[
            pl.BlockSpec((BB, EMBED_DIM), lambda b: (b, 0)),
            pl.BlockSpec((NUM_HEADS, NUM_EMB, HEAD_DIM), lambda b: (0, 0, 0)),
        ],
        out_specs=[
            pl.BlockSpec((QB, ROW_W), lambda b: (b, 0)),
            pl.BlockSpec((QB, ROW_W), lambda b: (b, 0)),
            pl.BlockSpec((QB, ROW_W), lambda b: (b, 0)),
            pl.BlockSpec((QB, ROW_W), lambda b: (b, 0)),
            pl.BlockSpec((NUM_HEADS, 1, BB), lambda b: (0, 0, b)),
            pl.BlockSpec((NUM_HEADS, 1, BB), lambda b: (0, 0, b)),
        ],
        out_shape=[
            quarter, quarter, quarter, quarter,
            jax.ShapeDtypeStruct((NUM_HEADS, 1, BATCH), jnp.int32),
            jax.ShapeDtypeStruct((NUM_HEADS, 1, BATCH), jnp.int32),
        ],
        compiler_params=pltpu.CompilerParams(
            dimension_semantics=("arbitrary",)),
    )(z_perm, cbn)


def _sc_join(d0, d1, d2, d3):
    info = plsc.get_sparse_core_info()
    nw = info.num_cores * info.num_subcores
    rows_per_w = BATCH // nw
    jsteps = rows_per_w // NQ
    nring = 3
    mesh = plsc.VectorSubcoreMesh(core_axis_name="c", subcore_axis_name="s")

    @functools.partial(
        pl.kernel,
        mesh=mesh,
        out_type=jax.ShapeDtypeStruct((BATCH, ROW_W), jnp.float32),
        scratch_types=[
            pltpu.VMEM((nring, ROW_W), jnp.float32),
            pltpu.SemaphoreType.DMA((nring,)),
            pltpu.SemaphoreType.DMA((nring,)),
        ],
        compiler_params=pltpu.CompilerParams(use_tc_tiling_on_sc=False),
    )
    def join_kernel(d0_hbm, d1_hbm, d2_hbm, d3_hbm, out_hbm,
                    buf, sem_r, sem_w):
        wid = lax.axis_index("s") * info.num_cores + lax.axis_index("c")
        out_base = wid * rows_per_w
        src_base = wid * jsteps
        srcs = (d0_hbm, d1_hbm, d2_hbm, d3_hbm)

        def read_dma(q, j, slot):
            return pltpu.make_async_copy(
                srcs[q].at[src_base + j, :], buf.at[slot], sem_r.at[slot])

        def write_dma(g, slot):
            return pltpu.make_async_copy(
                buf.at[slot], out_hbm.at[out_base + g, :], sem_w.at[slot])

        read_dma(0, 0, 0).start()

        def body(j, _):
            for q in range(NQ):
                g = NQ * j + q
                slot = lax.rem(g, nring)
                nslot = lax.rem(g + 1, nring)
                nq = (q + 1) % NQ
                nj = j + 1 if q == NQ - 1 else j

                @pl.when(g + 1 < rows_per_w)
                def _():
                    @pl.when(g + 1 >= nring)
                    def _():
                        write_dma(g + 1 - nring, nslot).wait()
                    read_dma(nq, nj, nslot).start()

                read_dma(q, j, slot).wait()
                write_dma(g, slot).start()
            return 0

        lax.fori_loop(0, jsteps, body, 0)
        for t in range(nring):
            g = rows_per_w - nring + t
            write_dma(g, lax.rem(g, nring)).wait()

    return join_kernel(d0, d1, d2, d3)


def _sc_gather(table, flat_idx):
    info = plsc.get_sparse_core_info()
    nw = info.num_cores * info.num_subcores
    rows = flat_idx.shape[0]
    per_w = rows // nw
    chunk = min(per_w, 1024)
    mesh = plsc.VectorSubcoreMesh(core_axis_name="c", subcore_axis_name="s")

    @functools.partial(
        pl.kernel,
        mesh=mesh,
        out_type=jax.ShapeDtypeStruct((rows, HEAD_DIM), jnp.float32),
        scratch_types=[
            pltpu.VMEM((chunk,), jnp.int32),
            pltpu.VMEM((chunk, HEAD_DIM), jnp.float32),
            pltpu.SemaphoreType.DMA,
        ],
        compiler_params=pltpu.CompilerParams(use_tc_tiling_on_sc=False),
    )
    def gather_kernel(table_hbm, fidx_hbm, out_hbm, idx_v, rows_v, sem):
        wid = lax.axis_index("s") * info.num_cores + lax.axis_index("c")
        base = wid * per_w
        for c in range(per_w // chunk):
            off = base + c * chunk
            pltpu.sync_copy(fidx_hbm.at[pl.ds(off, chunk)], idx_v)
            pltpu.async_copy(table_hbm.at[idx_v], rows_v, sem).wait()
            pltpu.sync_copy(rows_v, out_hbm.at[pl.ds(off, chunk)])

    return gather_kernel(table, flat_idx)


def kernel(z, codebooks):
    z_perm = z.reshape(BATCH // BB, QB, NQ, EMBED_DIM)
    z_perm = z_perm.transpose(0, 2, 1, 3).reshape(BATCH, EMBED_DIM)

    cbn = _normalize_cb(codebooks)
    d0, d1, d2, d3, idx_p, fidx_p = _dist_argmax(z_perm, cbn)

    dist2d = jnp.stack([d0, d1, d2, d3], axis=1).reshape(BATCH, ROW_W)
    distances = dist2d.reshape(BATCH, NUM_HEADS, NUM_EMB)

    def unperm(a):
        a = a.reshape(NUM_HEADS, BATCH // BB, NQ, QB)
        return a.transpose(0, 1, 3, 2).reshape(NUM_HEADS, BATCH)

    idx_hb = unperm(idx_p)
    fidx_hb = unperm(fidx_p)
    encoding_indices = idx_hb.T
    flat_idx = fidx_hb.T.reshape(-1)
    table = codebooks.reshape(NUM_HEADS * NUM_EMB, HEAD_DIM)
    zq = _sc_gather(table, flat_idx)
    z_q_st = zq.reshape(BATCH, EMBED_DIM)
    return (z_q_st, encoding_indices, distances)


# speedup vs baseline: 1.8915x; 1.8915x over previous
import functools

import jax
import jax.numpy as jnp
from jax import lax
from jax.experimental import pallas as pl
from jax.experimental.pallas import tpu as pltpu
from jax.experimental.pallas import tpu_sc as plsc

NUM_HEADS = 4
EMBED_DIM = 256
NUM_EMB = 8192
HEAD_DIM = EMBED_DIM // NUM_HEADS
BATCH = 16384

BB = 128


def _normalize_cb_body(cb_ref, cbn_ref):
    cb = cb_ref[...]
    cb_sq = jnp.sum(cb * cb, axis=-1, keepdims=True)
    cbn_ref[...] = cb / jnp.maximum(jnp.sqrt(cb_sq), 1e-12)


def _normalize_cb(codebooks):
    return pl.pallas_call(
        _normalize_cb_body,
        out_shape=jax.ShapeDtypeStruct(
            (NUM_HEADS, NUM_EMB, HEAD_DIM), jnp.float32),
    )(codebooks)


def _dist_argmax_body(z_ref, cbn_ref, dist_ref, idx_ref, fidx_ref):
    zb = z_ref[...]
    for h in range(NUM_HEADS):
        zh = zb[:, h * HEAD_DIM:(h + 1) * HEAD_DIM]
        zn_sq = jnp.sum(zh * zh, axis=-1, keepdims=True)
        zn = zh / jnp.maximum(jnp.sqrt(zn_sq), 1e-12)
        sims = lax.dot_general(
            zn, cbn_ref[h], (((1,), (1,)), ((), ())),
            preferred_element_type=jnp.float32)
        dist_ref[:, h * NUM_EMB:(h + 1) * NUM_EMB] = 1.0 - sims
        idx = jnp.argmax(sims, axis=-1).astype(jnp.int32)
        idx_ref[h, :] = idx
        fidx_ref[h, :] = idx + h * NUM_EMB


def _dist_argmax(z, cbn):
    grid = (BATCH // BB,)
    return pl.pallas_call(
        _dist_argmax_body,
        grid=grid,
        in_specs=[
            pl.BlockSpec((BB, EMBED_DIM), lambda b: (b, 0)),
            pl.BlockSpec((NUM_HEADS, NUM_EMB, HEAD_DIM), lambda b: (0, 0, 0)),
        ],
        out_specs=[
            pl.BlockSpec((BB, NUM_HEADS * NUM_EMB), lambda b: (b, 0)),
            pl.BlockSpec((NUM_HEADS, BB), lambda b: (0, b)),
            pl.BlockSpec((NUM_HEADS, BB), lambda b: (0, b)),
        ],
        out_shape=[
            jax.ShapeDtypeStruct((BATCH, NUM_HEADS * NUM_EMB), jnp.float32),
            jax.ShapeDtypeStruct((NUM_HEADS, BATCH), jnp.int32),
            jax.ShapeDtypeStruct((NUM_HEADS, BATCH), jnp.int32),
        ],
        compiler_params=pltpu.CompilerParams(
            dimension_semantics=("arbitrary",)),
    )(z, cbn)


def _sc_gather(table, flat_idx):
    info = plsc.get_sparse_core_info()
    nw = info.num_cores * info.num_subcores
    rows = flat_idx.shape[0]
    per_w = rows // nw
    chunk = min(per_w, 1024)
    mesh = plsc.VectorSubcoreMesh(core_axis_name="c", subcore_axis_name="s")

    @functools.partial(
        pl.kernel,
        mesh=mesh,
        out_type=jax.ShapeDtypeStruct((rows, HEAD_DIM), jnp.float32),
        scratch_types=[
            pltpu.VMEM((chunk,), jnp.int32),
            pltpu.VMEM((chunk, HEAD_DIM), jnp.float32),
            pltpu.SemaphoreType.DMA,
        ],
        compiler_params=pltpu.CompilerParams(use_tc_tiling_on_sc=False),
    )
    def gather_kernel(table_hbm, fidx_hbm, out_hbm, idx_v, rows_v, sem):
        wid = lax.axis_index("s") * info.num_cores + lax.axis_index("c")
        base = wid * per_w
        for c in range(per_w // chunk):
            off = base + c * chunk
            pltpu.sync_copy(fidx_hbm.at[pl.ds(off, chunk)], idx_v)
            pltpu.async_copy(table_hbm.at[idx_v], rows_v, sem).wait()
            pltpu.sync_copy(rows_v, out_hbm.at[pl.ds(off, chunk)])

    return gather_kernel(table, flat_idx)


def kernel(z, codebooks):
    cbn = _normalize_cb(codebooks)
    dist2d, idx_hb, fidx_hb = _dist_argmax(z, cbn)
    distances = dist2d.reshape(BATCH, NUM_HEADS, NUM_EMB)
    encoding_indices = idx_hb.T
    flat_idx = fidx_hb.T.reshape(-1)
    table = codebooks.reshape(NUM_HEADS * NUM_EMB, HEAD_DIM)
    zq = _sc_gather(table, flat_idx)
    z_q_st = zq.reshape(BATCH, EMBED_DIM)
    return (z_q_st, encoding_indices, distances)
